# baseline (speedup 1.0000x reference)
import functools

import jax
import jax.numpy as jnp
from jax import lax
from jax.experimental import pallas as pl
from jax.experimental.pallas import tpu as pltpu
from jax.experimental.pallas import tpu_sc as plsc


def _build_glove(B, V, D):
    info = plsc.get_sparse_core_info()
    NC, NS, L = info.num_cores, info.num_subcores, info.num_lanes
    NW = NC * NS
    BPW = B // NW
    CH = 128
    NCH = BPW // CH
    NGC = CH // L

    mesh = plsc.VectorSubcoreMesh(core_axis_name="c", subcore_axis_name="s")

    @functools.partial(
        pl.kernel,
        mesh=mesh,
        compiler_params=pltpu.CompilerParams(needs_layout_passes=False),
        out_type=jax.ShapeDtypeStruct((B,), jnp.float32),
        scratch_types=[
            pltpu.VMEM((NCH, CH), jnp.int32),
            pltpu.VMEM((NCH, CH), jnp.int32),
            pltpu.VMEM((2, 16, CH), jnp.int32),
            pltpu.VMEM((2, 16, CH), jnp.int32),
            pltpu.VMEM((2, 16, CH), jnp.float32),
            pltpu.VMEM((2, 16, CH), jnp.float32),
            pltpu.VMEM((BPW,), jnp.float32),
            pltpu.VMEM((BPW,), jnp.float32),
            pltpu.VMEM((BPW,), jnp.float32),
            pltpu.SemaphoreType.DMA,
            pltpu.SemaphoreType.DMA,
            pltpu.SemaphoreType.DMA,
        ],
    )
    def glove(ii_hbm, jj_hbm, wif_hbm, wjf_hbm, bi_hbm, bj_hbm, out_hbm,
              raw_i, raw_j, six_i, six_j, buf_i, buf_j, bv_i, bv_j, out_v,
              sem0, sem1, semb):
        wid = lax.axis_index("s") * NC + lax.axis_index("c")
        base = wid * BPW
        sems = (sem0, sem1)

        for c in range(NCH):
            pltpu.sync_copy(ii_hbm.at[pl.ds(base + c * CH, CH)], raw_i.at[c])
            pltpu.sync_copy(jj_hbm.at[pl.ds(base + c * CH, CH)], raw_j.at[c])

        bias_copies = []
        for c in range(NCH):
            sl = pl.ds(c * CH, CH)
            bias_copies.append(
                pltpu.async_copy(bi_hbm.at[raw_i.at[c]], bv_i.at[sl], semb))
            bias_copies.append(
                pltpu.async_copy(bj_hbm.at[raw_j.at[c]], bv_j.at[sl], semb))

        def fire(c):
            slot = c % 2
            def six_body(g, carry, c=c, slot=slot):
                sl = pl.ds(g * L, L)
                vi = raw_i[c, sl]
                vj = raw_j[c, sl]
                for d in range(D):
                    six_i[slot, d, sl] = vi + d * V
                    six_j[slot, d, sl] = vj + d * V
                return carry
            lax.fori_loop(0, NGC, six_body, 0)

            cps = []
            for d in range(D):
                cps.append(pltpu.async_copy(
                    wif_hbm.at[six_i.at[slot, d]], buf_i.at[slot, d],
                    sems[slot]))
                cps.append(pltpu.async_copy(
                    wjf_hbm.at[six_j.at[slot, d]], buf_j.at[slot, d],
                    sems[slot]))
            return cps

        row_copies = {0: fire(0)}

        for c in range(NCH):
            if c + 1 < NCH:
                row_copies[c + 1] = fire(c + 1)
            for cp in row_copies[c]:
                cp.wait()
            slot = c % 2

            def body(g, carry, c=c, slot=slot):
                sl = pl.ds(g * L, L)
                acc = jnp.zeros((L,), jnp.float32)
                for d in range(D):
                    acc = acc + buf_i[slot, d, sl] * buf_j[slot, d, sl]
                out_v[pl.ds(c * CH + g * L, L)] = acc
                return carry

            lax.fori_loop(0, NGC, body, 0)

        for cp in bias_copies:
            cp.wait()

        def bias_body(k, carry):
            sl = pl.ds(k * L, L)
            out_v[sl] = out_v[sl] + bv_i[sl] + bv_j[sl]
            return carry

        lax.fori_loop(0, BPW // L, bias_body, 0)

        pltpu.sync_copy(out_v, out_hbm.at[pl.ds(base, BPW)])

    return glove


def kernel(i_indices, j_indices, wi, wj, bi, bj):
    B = i_indices.shape[0]
    V, D = wi.shape
    glove = _build_glove(B, V, D)
    return glove(i_indices, j_indices,
                 wi.T.reshape(V * D), wj.T.reshape(V * D),
                 bi.reshape(V), bj.reshape(V))


# speedup vs baseline: 6.4890x; 6.4890x over previous
import functools

import jax
import jax.numpy as jnp
from jax import lax
from jax.experimental import pallas as pl
from jax.experimental.pallas import tpu as pltpu
from jax.experimental.pallas import tpu_sc as plsc

_BLK = 2048


def _tc_prep(wi_t, wj_t, bi_t, bj_t, V, D):
    n_steps = (V + _BLK - 1) // _BLK

    def body(wi_ref, wj_ref, bi_ref, bj_ref, *out_refs):
        yi_refs = out_refs[:D]
        yj_refs = out_refs[D:2 * D]
        bvi_ref, bvj_ref = out_refs[2 * D], out_refs[2 * D + 1]
        for d in range(D):
            yi_refs[d][...] = wi_ref[d, :]
            yj_refs[d][...] = wj_ref[d, :]
        bvi_ref[...] = bi_ref[0, :]
        bvj_ref[...] = bj_ref[0, :]

    table_spec = pl.BlockSpec((D, _BLK), lambda b: (0, b))
    bias_in_spec = pl.BlockSpec((1, _BLK), lambda b: (0, b))
    col_spec = pl.BlockSpec((_BLK,), lambda b: (b,))

    outs = pl.pallas_call(
        body,
        grid=(n_steps,),
        in_specs=[table_spec, table_spec, bias_in_spec, bias_in_spec],
        out_specs=[col_spec] * (2 * D + 2),
        out_shape=[jax.ShapeDtypeStruct((V,), jnp.float32)] * (2 * D + 2),
    )(wi_t, wj_t, bi_t, bj_t)
    return outs[:D], outs[D:2 * D], outs[2 * D], outs[2 * D + 1]


def _build_glove(B, V, D):
    info = plsc.get_sparse_core_info()
    NC, NS, L = info.num_cores, info.num_subcores, info.num_lanes
    NW = NC * NS
    BPW = B // NW
    CH = 128
    NCH = BPW // CH
    NGC = CH // L

    mesh = plsc.VectorSubcoreMesh(core_axis_name="c", subcore_axis_name="s")

    @functools.partial(
        pl.kernel,
        mesh=mesh,
        compiler_params=pltpu.CompilerParams(
            needs_layout_passes=False, use_tc_tiling_on_sc=False),
        out_type=jax.ShapeDtypeStruct((B,), jnp.float32),
        scratch_types=[
            pltpu.VMEM((NCH, CH), jnp.int32),
            pltpu.VMEM((NCH, CH), jnp.int32),
            pltpu.VMEM((2, 16, CH), jnp.float32),
            pltpu.VMEM((2, 16, CH), jnp.float32),
            pltpu.VMEM((BPW,), jnp.float32),
            pltpu.VMEM((BPW,), jnp.float32),
            pltpu.VMEM((BPW,), jnp.float32),
            pltpu.SemaphoreType.DMA,
            pltpu.SemaphoreType.DMA,
            pltpu.SemaphoreType.DMA,
        ],
    )
    def glove(ii_hbm, jj_hbm, *rest):
        yi_hbm = rest[:D]
        yj_hbm = rest[D:2 * D]
        bi_hbm, bj_hbm, out_hbm = rest[2 * D], rest[2 * D + 1], rest[2 * D + 2]
        (raw_i, raw_j, buf_i, buf_j, bv_i, bv_j, out_v,
         sem0, sem1, semb) = rest[2 * D + 3:]
        wid = lax.axis_index("s") * NC + lax.axis_index("c")
        base = wid * BPW
        sems = (sem0, sem1)

        for c in range(NCH):
            pltpu.sync_copy(ii_hbm.at[pl.ds(base + c * CH, CH)], raw_i.at[c])
            pltpu.sync_copy(jj_hbm.at[pl.ds(base + c * CH, CH)], raw_j.at[c])

        bias_copies = []
        for c in range(NCH):
            sl = pl.ds(c * CH, CH)
            bias_copies.append(
                pltpu.async_copy(bi_hbm.at[raw_i.at[c]], bv_i.at[sl], semb))
            bias_copies.append(
                pltpu.async_copy(bj_hbm.at[raw_j.at[c]], bv_j.at[sl], semb))

        def fire(c):
            slot = c % 2
            cps = []
            for d in range(D):
                cps.append(pltpu.async_copy(
                    yi_hbm[d].at[raw_i.at[c]], buf_i.at[slot, d],
                    sems[slot]))
                cps.append(pltpu.async_copy(
                    yj_hbm[d].at[raw_j.at[c]], buf_j.at[slot, d],
                    sems[slot]))
            return cps

        row_copies = {0: fire(0)}

        for c in range(NCH):
            if c + 1 < NCH:
                row_copies[c + 1] = fire(c + 1)
            for cp in row_copies[c]:
                cp.wait()
            slot = c % 2

            def body(g, carry, c=c, slot=slot):
                sl = pl.ds(g * L, L)
                acc = jnp.zeros((L,), jnp.float32)
                for d in range(D):
                    acc = acc + buf_i[slot, d, sl] * buf_j[slot, d, sl]
                out_v[pl.ds(c * CH + g * L, L)] = acc
                return carry

            lax.fori_loop(0, NGC, body, 0)

        for cp in bias_copies:
            cp.wait()

        def bias_body(k, carry):
            sl = pl.ds(k * L, L)
            out_v[sl] = out_v[sl] + bv_i[sl] + bv_j[sl]
            return carry

        lax.fori_loop(0, BPW // L, bias_body, 0)

        pltpu.sync_copy(out_v, out_hbm.at[pl.ds(base, BPW)])

    return glove


def kernel(i_indices, j_indices, wi, wj, bi, bj):
    B = i_indices.shape[0]
    V, D = wi.shape
    yi, yj, bvi, bvj = _tc_prep(wi.T, wj.T, bi.T, bj.T, V, D)
    glove = _build_glove(B, V, D)
    return glove(i_indices, j_indices, *yi, *yj, bvi, bvj)


# speedup vs baseline: 20.6955x; 3.1893x over previous
import functools

import jax
import jax.numpy as jnp
from jax import lax
from jax.experimental import pallas as pl
from jax.experimental.pallas import tpu as pltpu
from jax.experimental.pallas import tpu_sc as plsc


def _tc_prep(wi_t, wj_t, bi_t, bj_t, V, D):
    NBUF = 4
    NOUT = 2 * D + 2

    def body(wi_r, wj_r, bi_r, bj_r, *rest):
        yi_r = rest[:D]
        yj_r = rest[D:2 * D]
        bvi_r, bvj_r = rest[2 * D], rest[2 * D + 1]
        scratch = rest[NOUT:]
        bufs = scratch[:NBUF]
        semr = scratch[NBUF:2 * NBUF]
        semw = scratch[2 * NBUF:]

        jobs = []
        for d in range(D):
            jobs.append((wi_r.at[d], yi_r[d]))
            jobs.append((wj_r.at[d], yj_r[d]))
        jobs.append((bi_r.at[0], bvi_r))
        jobs.append((bj_r.at[0], bvj_r))

        K = len(jobs)
        rco = [None] * K
        wco = [None] * K
        for k in range(K):
            s = k % NBUF
            if k >= NBUF:
                wco[k - NBUF].wait()
            rco[k] = pltpu.async_copy(jobs[k][0], bufs[s], semr[s])
            if k >= 1:
                wco[k - 1] = pltpu.async_copy(
                    bufs[(k - 1) % NBUF], jobs[k - 1][1], semw[(k - 1) % NBUF])
                rco[k - 1].wait()
        rco[K - 1].wait()
        wco[K - 1] = pltpu.async_copy(
            bufs[(K - 1) % NBUF], jobs[K - 1][1], semw[(K - 1) % NBUF])
        for k in range(K - NBUF, K):
            wco[k].wait()

    any_spec = pl.BlockSpec(memory_space=pltpu.MemorySpace.HBM)
    outs = pl.pallas_call(
        body,
        in_specs=[any_spec] * 4,
        out_specs=[any_spec] * NOUT,
        out_shape=[jax.ShapeDtypeStruct((V,), jnp.float32)] * NOUT,
        scratch_shapes=(
            [pltpu.VMEM((V,), jnp.float32)] * NBUF
            + [pltpu.SemaphoreType.DMA] * (2 * NBUF)
        ),
    )(wi_t, wj_t, bi_t, bj_t)
    return outs[:D], outs[D:2 * D], outs[2 * D], outs[2 * D + 1]


def _build_glove(B, V, D):
    info = plsc.get_sparse_core_info()
    NC, NS, L = info.num_cores, info.num_subcores, info.num_lanes
    NW = NC * NS
    BPW = B // NW
    CH = 128
    NCH = BPW // CH
    NGC = CH // L

    mesh = plsc.VectorSubcoreMesh(core_axis_name="c", subcore_axis_name="s")

    @functools.partial(
        pl.kernel,
        mesh=mesh,
        compiler_params=pltpu.CompilerParams(
            needs_layout_passes=False, use_tc_tiling_on_sc=False),
        out_type=jax.ShapeDtypeStruct((B,), jnp.float32),
        scratch_types=[
            pltpu.VMEM((NCH, CH), jnp.int32),
            pltpu.VMEM((NCH, CH), jnp.int32),
            pltpu.VMEM((2, 16, CH), jnp.float32),
            pltpu.VMEM((2, 16, CH), jnp.float32),
            pltpu.VMEM((BPW,), jnp.float32),
            pltpu.VMEM((BPW,), jnp.float32),
            pltpu.VMEM((BPW,), jnp.float32),
            pltpu.SemaphoreType.DMA,
            pltpu.SemaphoreType.DMA,
            pltpu.SemaphoreType.DMA,
        ],
    )
    def glove(ii_hbm, jj_hbm, *rest):
        yi_hbm = rest[:D]
        yj_hbm = rest[D:2 * D]
        bi_hbm, bj_hbm, out_hbm = rest[2 * D], rest[2 * D + 1], rest[2 * D + 2]
        (raw_i, raw_j, buf_i, buf_j, bv_i, bv_j, out_v,
         sem0, sem1, semb) = rest[2 * D + 3:]
        wid = lax.axis_index("s") * NC + lax.axis_index("c")
        base = wid * BPW
        sems = (sem0, sem1)

        for c in range(NCH):
            pltpu.sync_copy(ii_hbm.at[pl.ds(base + c * CH, CH)], raw_i.at[c])
            pltpu.sync_copy(jj_hbm.at[pl.ds(base + c * CH, CH)], raw_j.at[c])

        bias_copies = []
        for c in range(NCH):
            sl = pl.ds(c * CH, CH)
            bias_copies.append(
                pltpu.async_copy(bi_hbm.at[raw_i.at[c]], bv_i.at[sl], semb))
            bias_copies.append(
                pltpu.async_copy(bj_hbm.at[raw_j.at[c]], bv_j.at[sl], semb))

        def fire(c):
            slot = c % 2
            cps = []
            for d in range(D):
                cps.append(pltpu.async_copy(
                    yi_hbm[d].at[raw_i.at[c]], buf_i.at[slot, d],
                    sems[slot]))
                cps.append(pltpu.async_copy(
                    yj_hbm[d].at[raw_j.at[c]], buf_j.at[slot, d],
                    sems[slot]))
            return cps

        row_copies = {0: fire(0)}

        for c in range(NCH):
            if c + 1 < NCH:
                row_copies[c + 1] = fire(c + 1)
            for cp in row_copies[c]:
                cp.wait()
            slot = c % 2

            def body(g, carry, c=c, slot=slot):
                sl = pl.ds(g * L, L)
                acc = jnp.zeros((L,), jnp.float32)
                for d in range(D):
                    acc = acc + buf_i[slot, d, sl] * buf_j[slot, d, sl]
                out_v[pl.ds(c * CH + g * L, L)] = acc
                return carry

            lax.fori_loop(0, NGC, body, 0)

        for cp in bias_copies:
            cp.wait()

        def bias_body(k, carry):
            sl = pl.ds(k * L, L)
            out_v[sl] = out_v[sl] + bv_i[sl] + bv_j[sl]
            return carry

        lax.fori_loop(0, BPW // L, bias_body, 0)

        pltpu.sync_copy(out_v, out_hbm.at[pl.ds(base, BPW)])

    return glove


def kernel(i_indices, j_indices, wi, wj, bi, bj):
    B = i_indices.shape[0]
    V, D = wi.shape
    yi, yj, bvi, bvj = _tc_prep(wi.T, wj.T, bi.T, bj.T, V, D)
    glove = _build_glove(B, V, D)
    return glove(i_indices, j_indices, *yi, *yj, bvi, bvj)


# speedup vs baseline: 20.7953x; 1.0048x over previous
import functools

import jax
import jax.numpy as jnp
from jax import lax
from jax.experimental import pallas as pl
from jax.experimental.pallas import tpu as pltpu
from jax.experimental.pallas import tpu_sc as plsc


def _tc_prep(wi_t, wj_t, bi_t, bj_t, V, D):
    NBUF = 4
    NOUT = 2 * D + 2

    def body(wi_r, wj_r, bi_r, bj_r, *rest):
        yi_r = rest[:D]
        yj_r = rest[D:2 * D]
        bvi_r, bvj_r = rest[2 * D], rest[2 * D + 1]
        scratch = rest[NOUT:]
        bufs = scratch[:NBUF]
        semr = scratch[NBUF:2 * NBUF]
        semw = scratch[2 * NBUF:]

        jobs = []
        for d in range(D):
            jobs.append((wi_r.at[d], yi_r[d]))
            jobs.append((wj_r.at[d], yj_r[d]))
        jobs.append((bi_r.at[0], bvi_r))
        jobs.append((bj_r.at[0], bvj_r))

        K = len(jobs)
        rco = [None] * K
        wco = [None] * K
        for k in range(K):
            s = k % NBUF
            if k >= NBUF:
                wco[k - NBUF].wait()
            rco[k] = pltpu.async_copy(jobs[k][0], bufs[s], semr[s])
            if k >= 1:
                rco[k - 1].wait()
                wco[k - 1] = pltpu.async_copy(
                    bufs[(k - 1) % NBUF], jobs[k - 1][1], semw[(k - 1) % NBUF])
        rco[K - 1].wait()
        wco[K - 1] = pltpu.async_copy(
            bufs[(K - 1) % NBUF], jobs[K - 1][1], semw[(K - 1) % NBUF])
        for k in range(K - NBUF, K):
            wco[k].wait()

    any_spec = pl.BlockSpec(memory_space=pltpu.MemorySpace.HBM)
    outs = pl.pallas_call(
        body,
        in_specs=[any_spec] * 4,
        out_specs=[any_spec] * NOUT,
        out_shape=[jax.ShapeDtypeStruct((V,), jnp.float32)] * NOUT,
        scratch_shapes=(
            [pltpu.VMEM((V,), jnp.float32)] * NBUF
            + [pltpu.SemaphoreType.DMA] * (2 * NBUF)
        ),
    )(wi_t, wj_t, bi_t, bj_t)
    return outs[:D], outs[D:2 * D], outs[2 * D], outs[2 * D + 1]


def _build_glove(B, V, D):
    info = plsc.get_sparse_core_info()
    NC, NS, L = info.num_cores, info.num_subcores, info.num_lanes
    NW = NC * NS
    BPW = B // NW
    CH = 128
    NCH = BPW // CH
    NGC = CH // L

    mesh = plsc.VectorSubcoreMesh(core_axis_name="c", subcore_axis_name="s")

    @functools.partial(
        pl.kernel,
        mesh=mesh,
        compiler_params=pltpu.CompilerParams(
            needs_layout_passes=False, use_tc_tiling_on_sc=False),
        out_type=jax.ShapeDtypeStruct((B,), jnp.float32),
        scratch_types=[
            pltpu.VMEM((NCH, CH), jnp.int32),
            pltpu.VMEM((NCH, CH), jnp.int32),
            pltpu.VMEM((2, 16, CH), jnp.float32),
            pltpu.VMEM((2, 16, CH), jnp.float32),
            pltpu.VMEM((BPW,), jnp.float32),
            pltpu.VMEM((BPW,), jnp.float32),
            pltpu.VMEM((BPW,), jnp.float32),
            pltpu.SemaphoreType.DMA,
            pltpu.SemaphoreType.DMA,
            pltpu.SemaphoreType.DMA,
        ],
    )
    def glove(ii_hbm, jj_hbm, *rest):
        yi_hbm = rest[:D]
        yj_hbm = rest[D:2 * D]
        bi_hbm, bj_hbm, out_hbm = rest[2 * D], rest[2 * D + 1], rest[2 * D + 2]
        (raw_i, raw_j, buf_i, buf_j, bv_i, bv_j, out_v,
         sem0, sem1, semb) = rest[2 * D + 3:]
        wid = lax.axis_index("s") * NC + lax.axis_index("c")
        base = wid * BPW
        sems = (sem0, sem1)

        for c in range(NCH):
            pltpu.sync_copy(ii_hbm.at[pl.ds(base + c * CH, CH)], raw_i.at[c])
            pltpu.sync_copy(jj_hbm.at[pl.ds(base + c * CH, CH)], raw_j.at[c])

        bias_copies = []
        for c in range(NCH):
            sl = pl.ds(c * CH, CH)
            bias_copies.append(
                pltpu.async_copy(bi_hbm.at[raw_i.at[c]], bv_i.at[sl], semb))
            bias_copies.append(
                pltpu.async_copy(bj_hbm.at[raw_j.at[c]], bv_j.at[sl], semb))

        def fire(c):
            slot = c % 2
            cps = []
            for d in range(D):
                cps.append(pltpu.async_copy(
                    yi_hbm[d].at[raw_i.at[c]], buf_i.at[slot, d],
                    sems[slot]))
                cps.append(pltpu.async_copy(
                    yj_hbm[d].at[raw_j.at[c]], buf_j.at[slot, d],
                    sems[slot]))
            return cps

        row_copies = {0: fire(0)}

        for c in range(NCH):
            if c + 1 < NCH:
                row_copies[c + 1] = fire(c + 1)
            for cp in row_copies[c]:
                cp.wait()
            slot = c % 2

            def body(g, carry, c=c, slot=slot):
                sl = pl.ds(g * L, L)
                acc = jnp.zeros((L,), jnp.float32)
                for d in range(D):
                    acc = acc + buf_i[slot, d, sl] * buf_j[slot, d, sl]
                out_v[pl.ds(c * CH + g * L, L)] = acc
                return carry

            lax.fori_loop(0, NGC, body, 0)

        for cp in bias_copies:
            cp.wait()

        def bias_body(k, carry):
            sl = pl.ds(k * L, L)
            out_v[sl] = out_v[sl] + bv_i[sl] + bv_j[sl]
            return carry

        lax.fori_loop(0, BPW // L, bias_body, 0)

        pltpu.sync_copy(out_v, out_hbm.at[pl.ds(base, BPW)])

    return glove


def kernel(i_indices, j_indices, wi, wj, bi, bj):
    B = i_indices.shape[0]
    V, D = wi.shape
    yi, yj, bvi, bvj = _tc_prep(wi.T, wj.T, bi.T, bj.T, V, D)
    glove = _build_glove(B, V, D)
    return glove(i_indices, j_indices, *yi, *yj, bvi, bvj)


# speedup vs baseline: 20.7961x; 1.0000x over previous
import functools

import jax
import jax.numpy as jnp
from jax import lax
from jax.experimental import pallas as pl
from jax.experimental.pallas import tpu as pltpu
from jax.experimental.pallas import tpu_sc as plsc


def _tc_prep(wi_t, wj_t, bi_t, bj_t, V, D):
    NBUF = 4
    NOUT = 2 * D + 2

    def body(wi_r, wj_r, bi_r, bj_r, *rest):
        yi_r = rest[:D]
        yj_r = rest[D:2 * D]
        bvi_r, bvj_r = rest[2 * D], rest[2 * D + 1]
        scratch = rest[NOUT:]
        bufs = scratch[:NBUF]
        semr = scratch[NBUF:2 * NBUF]
        semw = scratch[2 * NBUF:]

        jobs = []
        for d in range(D):
            jobs.append((wi_r.at[d], yi_r[d]))
            jobs.append((wj_r.at[d], yj_r[d]))
        jobs.append((bi_r.at[0], bvi_r))
        jobs.append((bj_r.at[0], bvj_r))

        K = len(jobs)
        rco = [None] * K
        wco = [None] * K
        for k in range(K):
            s = k % NBUF
            if k >= NBUF:
                wco[k - NBUF].wait()
            rco[k] = pltpu.async_copy(jobs[k][0], bufs[s], semr[s])
            if k >= 1:
                rco[k - 1].wait()
                wco[k - 1] = pltpu.async_copy(
                    bufs[(k - 1) % NBUF], jobs[k - 1][1], semw[(k - 1) % NBUF])
        rco[K - 1].wait()
        wco[K - 1] = pltpu.async_copy(
            bufs[(K - 1) % NBUF], jobs[K - 1][1], semw[(K - 1) % NBUF])
        for k in range(K - NBUF, K):
            wco[k].wait()

    any_spec = pl.BlockSpec(memory_space=pltpu.MemorySpace.HBM)
    outs = pl.pallas_call(
        body,
        in_specs=[any_spec] * 4,
        out_specs=[any_spec] * NOUT,
        out_shape=[jax.ShapeDtypeStruct((V,), jnp.float32)] * NOUT,
        scratch_shapes=(
            [pltpu.VMEM((V,), jnp.float32)] * NBUF
            + [pltpu.SemaphoreType.DMA] * (2 * NBUF)
        ),
    )(wi_t, wj_t, bi_t, bj_t)
    return outs[:D], outs[D:2 * D], outs[2 * D], outs[2 * D + 1]


def _build_glove(B, V, D):
    info = plsc.get_sparse_core_info()
    NC, NS, L = info.num_cores, info.num_subcores, info.num_lanes
    NW = NC * NS
    BPW = B // NW
    CH = 128
    NCH = BPW // CH
    NGC = CH // L

    mesh = plsc.VectorSubcoreMesh(core_axis_name="c", subcore_axis_name="s")

    @functools.partial(
        pl.kernel,
        mesh=mesh,
        compiler_params=pltpu.CompilerParams(
            needs_layout_passes=False, use_tc_tiling_on_sc=False),
        out_type=jax.ShapeDtypeStruct((B,), jnp.float32),
        scratch_types=[
            pltpu.VMEM((NCH, CH), jnp.int32),
            pltpu.VMEM((NCH, CH), jnp.int32),
            pltpu.VMEM((4, 16, CH), jnp.float32),
            pltpu.VMEM((4, 16, CH), jnp.float32),
            pltpu.VMEM((BPW,), jnp.float32),
            pltpu.VMEM((BPW,), jnp.float32),
            pltpu.VMEM((BPW,), jnp.float32),
            pltpu.SemaphoreType.DMA,
            pltpu.SemaphoreType.DMA,
            pltpu.SemaphoreType.DMA,
            pltpu.SemaphoreType.DMA,
            pltpu.SemaphoreType.DMA,
        ],
    )
    def glove(ii_hbm, jj_hbm, *rest):
        yi_hbm = rest[:D]
        yj_hbm = rest[D:2 * D]
        bi_hbm, bj_hbm, out_hbm = rest[2 * D], rest[2 * D + 1], rest[2 * D + 2]
        (raw_i, raw_j, buf_i, buf_j, bv_i, bv_j, out_v,
         sem0, sem1, sem2, sem3, semb) = rest[2 * D + 3:]
        wid = lax.axis_index("s") * NC + lax.axis_index("c")
        base = wid * BPW
        sems = (sem0, sem1, sem2, sem3)

        for c in range(NCH):
            pltpu.sync_copy(ii_hbm.at[pl.ds(base + c * CH, CH)], raw_i.at[c])
            pltpu.sync_copy(jj_hbm.at[pl.ds(base + c * CH, CH)], raw_j.at[c])

        bias_copies = []
        for c in range(NCH):
            sl = pl.ds(c * CH, CH)
            bias_copies.append(
                pltpu.async_copy(bi_hbm.at[raw_i.at[c]], bv_i.at[sl], semb))
            bias_copies.append(
                pltpu.async_copy(bj_hbm.at[raw_j.at[c]], bv_j.at[sl], semb))

        def fire(c):
            cps = []
            for d in range(D):
                cps.append(pltpu.async_copy(
                    yi_hbm[d].at[raw_i.at[c]], buf_i.at[c, d], sems[c]))
                cps.append(pltpu.async_copy(
                    yj_hbm[d].at[raw_j.at[c]], buf_j.at[c, d], sems[c]))
            return cps

        row_copies = {c: fire(c) for c in range(NCH)}

        for c in range(NCH):
            for cp in row_copies[c]:
                cp.wait()
            slot = c

            def body(g, carry, c=c, slot=slot):
                sl = pl.ds(g * L, L)
                acc = jnp.zeros((L,), jnp.float32)
                for d in range(D):
                    acc = acc + buf_i[slot, d, sl] * buf_j[slot, d, sl]
                out_v[pl.ds(c * CH + g * L, L)] = acc
                return carry

            lax.fori_loop(0, NGC, body, 0)

        for cp in bias_copies:
            cp.wait()

        def bias_body(k, carry):
            sl = pl.ds(k * L, L)
            out_v[sl] = out_v[sl] + bv_i[sl] + bv_j[sl]
            return carry

        lax.fori_loop(0, BPW // L, bias_body, 0)

        pltpu.sync_copy(out_v, out_hbm.at[pl.ds(base, BPW)])

    return glove


def kernel(i_indices, j_indices, wi, wj, bi, bj):
    B = i_indices.shape[0]
    V, D = wi.shape
    yi, yj, bvi, bvj = _tc_prep(wi.T, wj.T, bi.T, bj.T, V, D)
    glove = _build_glove(B, V, D)
    return glove(i_indices, j_indices, *yi, *yj, bvi, bvj)
